# Initial kernel scaffold; baseline (speedup 1.0000x reference)
import jax
import jax.numpy as jnp
from jax.experimental import pallas as pl


def kernel(pos, z, edge_index, W0, W1, W2, W_atom, b_atom, Wfc1, Wfc2):
    raise NotImplementedError("write your pallas kernel here")


# speedup vs baseline: 1.6187x; 1.6187x over previous
import functools
import math

import jax
import jax.numpy as jnp
from jax import lax
from jax.experimental import pallas as pl
from jax.experimental.pallas import tpu as pltpu
from jax.experimental.pallas import tpu_sc as plsc

N = 10000
E = 160000
NUM_BASIS = 32
MAX_RADIUS = 2.0
DIM = 288
HALF = 144
_SILU_CST = 1.6791767923989418

NPAD = 10240
_F32 = jnp.float32


def _atom_body(z_ref, wtA_ref, wtB_ref, bA_ref, bB_ref, oA_ref, oB_ref):
  zb = z_ref[...]
  oA_ref[...] = jnp.dot(zb, wtA_ref[...], preferred_element_type=_F32) + bA_ref[...]
  oB_ref[...] = jnp.dot(zb, wtB_ref[...], preferred_element_type=_F32) + bB_ref[...]


def _atom_tables(z, wtA, wtB, bA, bB):
  bl = 1000
  grid = N // bl
  return pl.pallas_call(
      _atom_body,
      grid=(grid,),
      in_specs=[
          pl.BlockSpec((bl, 4), lambda i: (i, 0)),
          pl.BlockSpec((4, HALF), lambda i: (0, 0)),
          pl.BlockSpec((4, HALF), lambda i: (0, 0)),
          pl.BlockSpec((1, HALF), lambda i: (0, 0)),
          pl.BlockSpec((1, HALF), lambda i: (0, 0)),
      ],
      out_specs=[
          pl.BlockSpec((bl, HALF), lambda i: (i, 0)),
          pl.BlockSpec((bl, HALF), lambda i: (i, 0)),
      ],
      out_shape=[
          jax.ShapeDtypeStruct((N, HALF), _F32),
          jax.ShapeDtypeStruct((N, HALF), _F32),
      ],
  )(z, wtA, wtB, bA, bB)


_EPT = E // 32
_VCH = 128
_VNCH = 39


def _vec_body(pos_x, pos_y, pos_z, srcs, dsts, vec_out,
              px, py, pz, sbuf, dbuf, obuf, sbt, dbt, obt):
  cid = lax.axis_index("c")
  sid = lax.axis_index("s")
  base = (cid * 16 + sid) * _EPT
  pltpu.sync_copy(pos_x, px)
  pltpu.sync_copy(pos_y, py)
  pltpu.sync_copy(pos_z, pz)
  iota = lax.iota(jnp.int32, 16)

  def groups(sb, db, ob, ngroups):
    for j in range(ngroups):
      si = sb[pl.ds(j * 16, 16)]
      di = db[pl.ds(j * 16, 16)]
      rows = iota + (j * 16)
      for comp, pref in ((0, px), (1, py), (2, pz)):
        a = plsc.load_gather(pref, [si])
        b = plsc.load_gather(pref, [di])
        plsc.store_scatter(ob, [rows, jnp.full((16,), comp, jnp.int32)], a - b)

  @pl.loop(0, _VNCH)
  def _(g):
    off = base + g * _VCH
    pltpu.sync_copy(srcs.at[pl.ds(off, _VCH)], sbuf)
    pltpu.sync_copy(dsts.at[pl.ds(off, _VCH)], dbuf)
    groups(sbuf, dbuf, obuf, 8)
    pltpu.sync_copy(obuf, vec_out.at[pl.ds(off, _VCH)])

  toff = base + _EPT - 16
  pltpu.sync_copy(srcs.at[pl.ds(toff, 16)], sbt)
  pltpu.sync_copy(dsts.at[pl.ds(toff, 16)], dbt)
  groups(sbt, dbt, obt, 1)
  pltpu.sync_copy(obt, vec_out.at[pl.ds(toff, 16)])


def _edge_vec(pos_x, pos_y, pos_z, srcs, dsts):
  mesh = plsc.VectorSubcoreMesh(core_axis_name="c", subcore_axis_name="s")
  f = pl.kernel(
      _vec_body,
      out_type=jax.ShapeDtypeStruct((E, 3), _F32),
      mesh=mesh,
      scratch_types=[
          pltpu.VMEM((N,), _F32),
          pltpu.VMEM((N,), _F32),
          pltpu.VMEM((N,), _F32),
          pltpu.VMEM((_VCH,), jnp.int32),
          pltpu.VMEM((_VCH,), jnp.int32),
          pltpu.VMEM((_VCH, 3), _F32),
          pltpu.VMEM((16,), jnp.int32),
          pltpu.VMEM((16,), jnp.int32),
          pltpu.VMEM((16, 3), _F32),
      ],
      compiler_params=pltpu.CompilerParams(needs_layout_passes=False),
  )
  return f(pos_x, pos_y, pos_z, srcs, dsts)


_EBL = 640


def _dense_body(vec_ref, StA_ref, StB_ref, W1s_ref, W2sA_ref, W2sB_ref,
                oA_ref, oB_ref):
  v = vec_ref[...]
  x = v[:, 0:1]
  y = v[:, 1:2]
  zc = v[:, 2:3]
  r2 = x * x + y * y + zc * zc
  vl = jnp.sqrt(r2 + 1e-18)
  ux = x / vl
  uy = y / vl
  uz = zc / vl
  s3 = math.sqrt(3.0)
  s5 = math.sqrt(5.0)
  x2 = ux * ux
  y2 = uy * uy
  z2 = uz * uz
  sh9 = jnp.concatenate([
      jnp.ones_like(vl),
      s3 * ux, s3 * uy, s3 * uz,
      s5 * (s3 * ux * uz),
      s5 * (s3 * ux * uy),
      s5 * (y2 - 0.5 * (x2 + z2)),
      s5 * (s3 * uy * uz),
      s5 * (0.5 * s3 * (z2 - x2)),
  ], axis=1)
  roots = (lax.broadcasted_iota(jnp.int32, (1, NUM_BASIS), 1).astype(_F32)
           + 1.0) * math.pi
  safe_r = jnp.where(vl > 1e-9, vl, 1.0)
  rb = jnp.sqrt(2.0 / MAX_RADIUS) * jnp.sin(roots * vl / MAX_RADIUS) / safe_r
  mask = ((vl < MAX_RADIUS) & (vl > 0)).astype(_F32)
  rb = rb * mask
  h = _SILU_CST * jax.nn.silu(
      jnp.dot(rb, W1s_ref[...], preferred_element_type=_F32))
  oA_ref[...] = (jnp.dot(h, W2sA_ref[...], preferred_element_type=_F32) *
                 jnp.dot(sh9, StA_ref[...], preferred_element_type=_F32))
  oB_ref[...] = (jnp.dot(h, W2sB_ref[...], preferred_element_type=_F32) *
                 jnp.dot(sh9, StB_ref[...], preferred_element_type=_F32))


def _edge_dense(vec, StA, StB, W1s, W2sA, W2sB):
  grid = E // _EBL
  return pl.pallas_call(
      _dense_body,
      grid=(grid,),
      in_specs=[
          pl.BlockSpec((_EBL, 3), lambda i: (i, 0)),
          pl.BlockSpec((9, HALF), lambda i: (0, 0)),
          pl.BlockSpec((9, HALF), lambda i: (0, 0)),
          pl.BlockSpec((NUM_BASIS, NUM_BASIS), lambda i: (0, 0)),
          pl.BlockSpec((NUM_BASIS, HALF), lambda i: (0, 0)),
          pl.BlockSpec((NUM_BASIS, HALF), lambda i: (0, 0)),
      ],
      out_specs=[
          pl.BlockSpec((_EBL, HALF), lambda i: (i, 0)),
          pl.BlockSpec((_EBL, HALF), lambda i: (i, 0)),
      ],
      out_shape=[
          jax.ShapeDtypeStruct((E, HALF), _F32),
          jax.ShapeDtypeStruct((E, HALF), _F32),
      ],
  )(vec, StA, StB, W1s, W2sA, W2sB)


_K = 40
_NCH = (E // 16) // _K
_EPS = E // 16


def _scatter_body(atomA, atomB, radA, radB, srcs, dsts, outA, outB,
                  acc, zb,
                  ss0, ds0, gb0, rb0, ss1, ds1, gb1, rb1,
                  sg0, sr0, sg1, sr1):
  cid = lax.axis_index("c")
  sid = lax.axis_index("s")

  @pl.loop(0, _K)
  def _(i):
    for k in range(HALF // 16):
      zb[i, pl.ds(k * 16, 16)] = jnp.zeros((16,), _F32)

  @pl.loop(0, NPAD // 16 // _K)
  def _(t):
    pltpu.sync_copy(zb, acc.at[pl.ds(sid * (NPAD // 16) + t * _K, _K)])
  plsc.subcore_barrier()

  base = sid * _EPS
  bufs = ((ss0, ds0, gb0, rb0, sg0, sr0), (ss1, ds1, gb1, rb1, sg1, sr1))

  def run(at_, rd_, ot_):
    def issue(g, bk):
      ss, dd, gb, rbf, sg, sr = bk
      off = base + g * _K
      pltpu.sync_copy(srcs.at[pl.ds(off, _K)], ss)
      pltpu.sync_copy(dsts.at[pl.ds(off, _K)], dd)
      pltpu.make_async_copy(at_.at[ss], gb, sg).start()
      pltpu.make_async_copy(rd_.at[pl.ds(off, _K)], rbf, sr).start()

    def process(g, bk):
      ss, dd, gb, rbf, sg, sr = bk
      off = base + g * _K
      pltpu.make_async_copy(at_.at[ss], gb, sg).wait()
      pltpu.make_async_copy(rd_.at[pl.ds(off, _K)], rbf, sr).wait()

      @pl.loop(0, _K)
      def _(i):
        for k in range(HALF // 16):
          sl = pl.ds(k * 16, 16)
          gb[i, sl] = gb[i, sl] * rbf[i, sl]

      pltpu.sync_copy(gb, acc.at[dd], add=True)

    issue(0, bufs[0])

    @pl.loop(0, _NCH - 2, step=2)
    def _(t):
      for b in range(2):
        g = t + b
        issue(g + 1, bufs[1 - b])
        process(g, bufs[b])

    issue(_NCH - 1, bufs[(_NCH - 1) % 2])
    process(_NCH - 2, bufs[(_NCH - 2) % 2])
    process(_NCH - 1, bufs[(_NCH - 1) % 2])
    plsc.subcore_barrier()

    @pl.loop(0, NPAD // 16 // _K)
    def _(t):
      row = sid * (NPAD // 16) + t * _K
      pltpu.sync_copy(acc.at[pl.ds(row, _K)], zb)
      pltpu.sync_copy(zb, ot_.at[pl.ds(row, _K)])

  @pl.when(cid == 0)
  def _():
    run(atomA, radA, outA)

  @pl.when(cid == 1)
  def _():
    run(atomB, radB, outB)


def _scatter(atomA, atomB, radA, radB, srcs, dsts):
  mesh = plsc.VectorSubcoreMesh(core_axis_name="c", subcore_axis_name="s")
  f = pl.kernel(
      _scatter_body,
      out_type=(
          jax.ShapeDtypeStruct((NPAD, HALF), _F32),
          jax.ShapeDtypeStruct((NPAD, HALF), _F32),
      ),
      mesh=mesh,
      scratch_types=[
          pltpu.VMEM_SHARED((NPAD, HALF), _F32),
          pltpu.VMEM((_K, HALF), _F32),
          pltpu.VMEM((_K,), jnp.int32),
          pltpu.VMEM((_K,), jnp.int32),
          pltpu.VMEM((_K, HALF), _F32),
          pltpu.VMEM((_K, HALF), _F32),
          pltpu.VMEM((_K,), jnp.int32),
          pltpu.VMEM((_K,), jnp.int32),
          pltpu.VMEM((_K, HALF), _F32),
          pltpu.VMEM((_K, HALF), _F32),
          pltpu.SemaphoreType.DMA,
          pltpu.SemaphoreType.DMA,
          pltpu.SemaphoreType.DMA,
          pltpu.SemaphoreType.DMA,
      ],
      compiler_params=pltpu.CompilerParams(use_tc_tiling_on_sc=False),
  )
  return f(atomA, atomB, radA, radB, srcs, dsts)


def kernel(pos, z, edge_index, W0, W1, W2, W_atom, b_atom, Wfc1, Wfc2):
  srcs = edge_index[0]
  dsts = edge_index[1]
  pos_x, pos_y, pos_z = pos[:, 0], pos[:, 1], pos[:, 2]

  St = jnp.zeros((9, DIM), _F32)
  St = St.at[0, 0:32].set(W0)
  St = St.at[1:4, 32:128].set(jnp.kron(W1[None, :], jnp.eye(3, dtype=_F32)))
  St = St.at[4:9, 128:288].set(jnp.kron(W2[None, :], jnp.eye(5, dtype=_F32)))
  StA, StB = St[:, :HALF], St[:, HALF:]
  W1s = Wfc1 / math.sqrt(float(NUM_BASIS))
  W2s = Wfc2 / math.sqrt(32.0)
  W2sA, W2sB = W2s[:, :HALF], W2s[:, HALF:]
  wt = W_atom.T
  wtA, wtB = wt[:, :HALF], wt[:, HALF:]
  bA, bB = b_atom[None, :HALF], b_atom[None, HALF:]

  atomA, atomB = _atom_tables(z, wtA, wtB, bA, bB)
  vec = _edge_vec(pos_x, pos_y, pos_z, srcs, dsts)
  radA, radB = _edge_dense(vec, StA, StB, W1s, W2sA, W2sB)
  outA, outB = _scatter(atomA, atomB, radA, radB, srcs, dsts)
  return jnp.concatenate([outA[:N], outB[:N]], axis=1)


# speedup vs baseline: 2.4464x; 1.5114x over previous
import functools
import math

import jax
import jax.numpy as jnp
from jax import lax
from jax.experimental import pallas as pl
from jax.experimental.pallas import tpu as pltpu
from jax.experimental.pallas import tpu_sc as plsc

N = 10000
E = 160000
NUM_BASIS = 32
MAX_RADIUS = 2.0
DIM = 288
HALF = 144
_SILU_CST = 1.6791767923989418

NPAD = 10240
_F32 = jnp.float32


def _atom_body(z_ref, wtA_ref, wtB_ref, bA_ref, bB_ref, oA_ref, oB_ref):
  zb = z_ref[...]
  oA_ref[...] = jnp.dot(zb, wtA_ref[...], preferred_element_type=_F32) + bA_ref[...]
  oB_ref[...] = jnp.dot(zb, wtB_ref[...], preferred_element_type=_F32) + bB_ref[...]


def _atom_tables(z, wtA, wtB, bA, bB):
  bl = 1000
  grid = N // bl
  return pl.pallas_call(
      _atom_body,
      grid=(grid,),
      in_specs=[
          pl.BlockSpec((bl, 4), lambda i: (i, 0)),
          pl.BlockSpec((4, HALF), lambda i: (0, 0)),
          pl.BlockSpec((4, HALF), lambda i: (0, 0)),
          pl.BlockSpec((1, HALF), lambda i: (0, 0)),
          pl.BlockSpec((1, HALF), lambda i: (0, 0)),
      ],
      out_specs=[
          pl.BlockSpec((bl, HALF), lambda i: (i, 0)),
          pl.BlockSpec((bl, HALF), lambda i: (i, 0)),
      ],
      out_shape=[
          jax.ShapeDtypeStruct((N, HALF), _F32),
          jax.ShapeDtypeStruct((N, HALF), _F32),
      ],
  )(z, wtA, wtB, bA, bB)


_EPT = E // 32


def _vec_body(pos_x, pos_y, pos_z, srcs, dsts, vx_o, vy_o, vz_o,
              px, py, pz, sv, dv, ox, oy, oz):
  cid = lax.axis_index("c")
  sid = lax.axis_index("s")
  base = (cid * 16 + sid) * _EPT
  pltpu.sync_copy(pos_x, px)
  pltpu.sync_copy(pos_y, py)
  pltpu.sync_copy(pos_z, pz)
  pltpu.sync_copy(srcs.at[pl.ds(base, _EPT)], sv)
  pltpu.sync_copy(dsts.at[pl.ds(base, _EPT)], dv)

  def group(off):
    si = sv[pl.ds(off, 16)]
    di = dv[pl.ds(off, 16)]
    for pref, ob in ((px, ox), (py, oy), (pz, oz)):
      a = plsc.load_gather(pref, [si])
      b = plsc.load_gather(pref, [di])
      ob[pl.ds(off, 16)] = a - b

  @pl.loop(0, _EPT // 16)
  def _(g):
    group(g * 16)

  group(_EPT - 16)

  pltpu.sync_copy(ox, vx_o.at[pl.ds(base, _EPT)])
  pltpu.sync_copy(oy, vy_o.at[pl.ds(base, _EPT)])
  pltpu.sync_copy(oz, vz_o.at[pl.ds(base, _EPT)])


def _edge_vec(pos_x, pos_y, pos_z, srcs, dsts):
  mesh = plsc.VectorSubcoreMesh(core_axis_name="c", subcore_axis_name="s")
  f = pl.kernel(
      _vec_body,
      out_type=(
          jax.ShapeDtypeStruct((E,), _F32),
          jax.ShapeDtypeStruct((E,), _F32),
          jax.ShapeDtypeStruct((E,), _F32),
      ),
      mesh=mesh,
      scratch_types=[
          pltpu.VMEM((N,), _F32),
          pltpu.VMEM((N,), _F32),
          pltpu.VMEM((N,), _F32),
          pltpu.VMEM((_EPT,), jnp.int32),
          pltpu.VMEM((_EPT,), jnp.int32),
          pltpu.VMEM((_EPT,), _F32),
          pltpu.VMEM((_EPT,), _F32),
          pltpu.VMEM((_EPT,), _F32),
      ],
      compiler_params=pltpu.CompilerParams(needs_layout_passes=False),
  )
  return f(pos_x, pos_y, pos_z, srcs, dsts)


_EBL = 640


def _dense_body(vx_ref, vy_ref, vz_ref, StAT_ref, StBT_ref, W1sT_ref,
                W2sAT_ref, W2sBT_ref, oA_ref, oB_ref):
  x = vx_ref[0]
  y = vy_ref[0]
  zc = vz_ref[0]
  vl = jnp.sqrt(x * x + y * y + zc * zc + 1e-18)
  inv = 1.0 / vl
  ux = x * inv
  uy = y * inv
  uz = zc * inv
  s3 = math.sqrt(3.0)
  s5 = math.sqrt(5.0)
  x2 = ux * ux
  y2 = uy * uy
  z2 = uz * uz
  sh9T = jnp.concatenate([
      jnp.ones_like(vl),
      s3 * ux, s3 * uy, s3 * uz,
      s5 * (s3 * ux * uz),
      s5 * (s3 * ux * uy),
      s5 * (y2 - 0.5 * (x2 + z2)),
      s5 * (s3 * uy * uz),
      s5 * (0.5 * s3 * (z2 - x2)),
  ], axis=0)
  vlT = vl
  roots = ((lax.broadcasted_iota(jnp.int32, (NUM_BASIS, 1), 0).astype(_F32)
            + 1.0) * math.pi)
  safe_r = jnp.where(vlT > 1e-9, vlT, 1.0)
  mask = ((vlT < MAX_RADIUS) & (vlT > 0)).astype(_F32)
  coef = jnp.sqrt(2.0 / MAX_RADIUS) * mask / safe_r
  rbT = jnp.sin(roots * (vlT / MAX_RADIUS)) * coef
  hT = _SILU_CST * jax.nn.silu(
      jnp.dot(W1sT_ref[...], rbT, preferred_element_type=_F32))
  outAT = (jnp.dot(W2sAT_ref[...], hT, preferred_element_type=_F32) *
           jnp.dot(StAT_ref[...], sh9T, preferred_element_type=_F32))
  outBT = (jnp.dot(W2sBT_ref[...], hT, preferred_element_type=_F32) *
           jnp.dot(StBT_ref[...], sh9T, preferred_element_type=_F32))
  oA_ref[...] = outAT.T
  oB_ref[...] = outBT.T


def _edge_dense(vx, vy, vz, StAT, StBT, W1sT, W2sAT, W2sBT):
  grid = E // _EBL
  vx = vx.reshape(grid, 1, _EBL)
  vy = vy.reshape(grid, 1, _EBL)
  vz = vz.reshape(grid, 1, _EBL)
  return pl.pallas_call(
      _dense_body,
      grid=(grid,),
      in_specs=[
          pl.BlockSpec((1, 1, _EBL), lambda i: (i, 0, 0)),
          pl.BlockSpec((1, 1, _EBL), lambda i: (i, 0, 0)),
          pl.BlockSpec((1, 1, _EBL), lambda i: (i, 0, 0)),
          pl.BlockSpec((HALF, 9), lambda i: (0, 0)),
          pl.BlockSpec((HALF, 9), lambda i: (0, 0)),
          pl.BlockSpec((NUM_BASIS, NUM_BASIS), lambda i: (0, 0)),
          pl.BlockSpec((HALF, NUM_BASIS), lambda i: (0, 0)),
          pl.BlockSpec((HALF, NUM_BASIS), lambda i: (0, 0)),
      ],
      out_specs=[
          pl.BlockSpec((_EBL, HALF), lambda i: (i, 0)),
          pl.BlockSpec((_EBL, HALF), lambda i: (i, 0)),
      ],
      out_shape=[
          jax.ShapeDtypeStruct((E, HALF), _F32),
          jax.ShapeDtypeStruct((E, HALF), _F32),
      ],
  )(vx, vy, vz, StAT, StBT, W1sT, W2sAT, W2sBT)


_K = 40
_NCH = (E // 16) // _K
_EPS = E // 16


def _scatter_body(atomA, atomB, radA, radB, srcs, dsts, outA, outB,
                  acc, zb,
                  ss0, ds0, gb0, rb0, ss1, ds1, gb1, rb1,
                  sg0, sr0, sg1, sr1):
  cid = lax.axis_index("c")
  sid = lax.axis_index("s")

  @pl.loop(0, _K)
  def _(i):
    for k in range(HALF // 16):
      zb[i, pl.ds(k * 16, 16)] = jnp.zeros((16,), _F32)

  @pl.loop(0, NPAD // 16 // _K)
  def _(t):
    pltpu.sync_copy(zb, acc.at[pl.ds(sid * (NPAD // 16) + t * _K, _K)])
  plsc.subcore_barrier()

  base = sid * _EPS
  bufs = ((ss0, ds0, gb0, rb0, sg0, sr0), (ss1, ds1, gb1, rb1, sg1, sr1))

  def run(at_, rd_, ot_):
    def issue(g, bk):
      ss, dd, gb, rbf, sg, sr = bk
      off = base + g * _K
      pltpu.sync_copy(srcs.at[pl.ds(off, _K)], ss)
      pltpu.sync_copy(dsts.at[pl.ds(off, _K)], dd)
      pltpu.make_async_copy(at_.at[ss], gb, sg).start()
      pltpu.make_async_copy(rd_.at[pl.ds(off, _K)], rbf, sr).start()

    def process(g, bk):
      ss, dd, gb, rbf, sg, sr = bk
      off = base + g * _K
      pltpu.make_async_copy(at_.at[ss], gb, sg).wait()
      pltpu.make_async_copy(rd_.at[pl.ds(off, _K)], rbf, sr).wait()

      @pl.loop(0, _K)
      def _(i):
        for k in range(HALF // 16):
          sl = pl.ds(k * 16, 16)
          gb[i, sl] = gb[i, sl] * rbf[i, sl]

      pltpu.sync_copy(gb, acc.at[dd], add=True)

    issue(0, bufs[0])

    @pl.loop(0, _NCH - 2, step=2)
    def _(t):
      for b in range(2):
        g = t + b
        issue(g + 1, bufs[1 - b])
        process(g, bufs[b])

    issue(_NCH - 1, bufs[(_NCH - 1) % 2])
    process(_NCH - 2, bufs[(_NCH - 2) % 2])
    process(_NCH - 1, bufs[(_NCH - 1) % 2])
    plsc.subcore_barrier()

    @pl.loop(0, NPAD // 16 // _K)
    def _(t):
      row = sid * (NPAD // 16) + t * _K
      pltpu.sync_copy(acc.at[pl.ds(row, _K)], zb)
      pltpu.sync_copy(zb, ot_.at[pl.ds(row, _K)])

  @pl.when(cid == 0)
  def _():
    run(atomA, radA, outA)

  @pl.when(cid == 1)
  def _():
    run(atomB, radB, outB)


def _scatter(atomA, atomB, radA, radB, srcs, dsts):
  mesh = plsc.VectorSubcoreMesh(core_axis_name="c", subcore_axis_name="s")
  f = pl.kernel(
      _scatter_body,
      out_type=(
          jax.ShapeDtypeStruct((NPAD, HALF), _F32),
          jax.ShapeDtypeStruct((NPAD, HALF), _F32),
      ),
      mesh=mesh,
      scratch_types=[
          pltpu.VMEM_SHARED((NPAD, HALF), _F32),
          pltpu.VMEM((_K, HALF), _F32),
          pltpu.VMEM((_K,), jnp.int32),
          pltpu.VMEM((_K,), jnp.int32),
          pltpu.VMEM((_K, HALF), _F32),
          pltpu.VMEM((_K, HALF), _F32),
          pltpu.VMEM((_K,), jnp.int32),
          pltpu.VMEM((_K,), jnp.int32),
          pltpu.VMEM((_K, HALF), _F32),
          pltpu.VMEM((_K, HALF), _F32),
          pltpu.SemaphoreType.DMA,
          pltpu.SemaphoreType.DMA,
          pltpu.SemaphoreType.DMA,
          pltpu.SemaphoreType.DMA,
      ],
      compiler_params=pltpu.CompilerParams(use_tc_tiling_on_sc=False),
  )
  return f(atomA, atomB, radA, radB, srcs, dsts)


def kernel(pos, z, edge_index, W0, W1, W2, W_atom, b_atom, Wfc1, Wfc2):
  srcs = edge_index[0]
  dsts = edge_index[1]
  pos_x, pos_y, pos_z = pos[:, 0], pos[:, 1], pos[:, 2]

  St = jnp.zeros((9, DIM), _F32)
  St = St.at[0, 0:32].set(W0)
  St = St.at[1:4, 32:128].set(jnp.kron(W1[None, :], jnp.eye(3, dtype=_F32)))
  St = St.at[4:9, 128:288].set(jnp.kron(W2[None, :], jnp.eye(5, dtype=_F32)))
  StAT, StBT = St[:, :HALF].T, St[:, HALF:].T
  W1sT = (Wfc1 / math.sqrt(float(NUM_BASIS))).T
  W2s = Wfc2 / math.sqrt(32.0)
  W2sAT, W2sBT = W2s[:, :HALF].T, W2s[:, HALF:].T
  wt = W_atom.T
  wtA, wtB = wt[:, :HALF], wt[:, HALF:]
  bA, bB = b_atom[None, :HALF], b_atom[None, HALF:]

  atomA, atomB = _atom_tables(z, wtA, wtB, bA, bB)
  vx, vy, vz = _edge_vec(pos_x, pos_y, pos_z, srcs, dsts)
  radA, radB = _edge_dense(vx, vy, vz, StAT, StBT, W1sT, W2sAT, W2sBT)
  outA, outB = _scatter(atomA, atomB, radA, radB, srcs, dsts)
  return jnp.concatenate([outA[:N], outB[:N]], axis=1)


# speedup vs baseline: 2.5700x; 1.0505x over previous
import functools
import math

import jax
import jax.numpy as jnp
from jax import lax
from jax.experimental import pallas as pl
from jax.experimental.pallas import tpu as pltpu
from jax.experimental.pallas import tpu_sc as plsc

N = 10000
E = 160000
NUM_BASIS = 32
MAX_RADIUS = 2.0
DIM = 288
HALF = 144
_SILU_CST = 1.6791767923989418

NPAD = 10240
_F32 = jnp.float32


def _atom_body(z_ref, wtA_ref, wtB_ref, bA_ref, bB_ref, oA_ref, oB_ref):
  zb = z_ref[...]
  oA_ref[...] = jnp.dot(zb, wtA_ref[...], preferred_element_type=_F32) + bA_ref[...]
  oB_ref[...] = jnp.dot(zb, wtB_ref[...], preferred_element_type=_F32) + bB_ref[...]


def _atom_tables(z, wtA, wtB, bA, bB):
  bl = 1000
  grid = N // bl
  return pl.pallas_call(
      _atom_body,
      grid=(grid,),
      in_specs=[
          pl.BlockSpec((bl, 4), lambda i: (i, 0)),
          pl.BlockSpec((4, HALF), lambda i: (0, 0)),
          pl.BlockSpec((4, HALF), lambda i: (0, 0)),
          pl.BlockSpec((1, HALF), lambda i: (0, 0)),
          pl.BlockSpec((1, HALF), lambda i: (0, 0)),
      ],
      out_specs=[
          pl.BlockSpec((bl, HALF), lambda i: (i, 0)),
          pl.BlockSpec((bl, HALF), lambda i: (i, 0)),
      ],
      out_shape=[
          jax.ShapeDtypeStruct((N, HALF), _F32),
          jax.ShapeDtypeStruct((N, HALF), _F32),
      ],
  )(z, wtA, wtB, bA, bB)


_EPT = E // 32


def _vec_body(pos_x, pos_y, pos_z, srcs, dsts, vx_o, vy_o, vz_o,
              px, py, pz, sv, dv, ox, oy, oz):
  cid = lax.axis_index("c")
  sid = lax.axis_index("s")
  base = (cid * 16 + sid) * _EPT
  pltpu.sync_copy(pos_x, px)
  pltpu.sync_copy(pos_y, py)
  pltpu.sync_copy(pos_z, pz)
  pltpu.sync_copy(srcs.at[pl.ds(base, _EPT)], sv)
  pltpu.sync_copy(dsts.at[pl.ds(base, _EPT)], dv)

  def group(off):
    si = sv[pl.ds(off, 16)]
    di = dv[pl.ds(off, 16)]
    for pref, ob in ((px, ox), (py, oy), (pz, oz)):
      a = plsc.load_gather(pref, [si])
      b = plsc.load_gather(pref, [di])
      ob[pl.ds(off, 16)] = a - b

  @pl.loop(0, _EPT // 16)
  def _(g):
    group(g * 16)

  group(_EPT - 16)

  pltpu.sync_copy(ox, vx_o.at[pl.ds(base, _EPT)])
  pltpu.sync_copy(oy, vy_o.at[pl.ds(base, _EPT)])
  pltpu.sync_copy(oz, vz_o.at[pl.ds(base, _EPT)])


def _edge_vec(pos_x, pos_y, pos_z, srcs, dsts):
  mesh = plsc.VectorSubcoreMesh(core_axis_name="c", subcore_axis_name="s")
  f = pl.kernel(
      _vec_body,
      out_type=(
          jax.ShapeDtypeStruct((E,), _F32),
          jax.ShapeDtypeStruct((E,), _F32),
          jax.ShapeDtypeStruct((E,), _F32),
      ),
      mesh=mesh,
      scratch_types=[
          pltpu.VMEM((N,), _F32),
          pltpu.VMEM((N,), _F32),
          pltpu.VMEM((N,), _F32),
          pltpu.VMEM((_EPT,), jnp.int32),
          pltpu.VMEM((_EPT,), jnp.int32),
          pltpu.VMEM((_EPT,), _F32),
          pltpu.VMEM((_EPT,), _F32),
          pltpu.VMEM((_EPT,), _F32),
      ],
      compiler_params=pltpu.CompilerParams(needs_layout_passes=False),
  )
  return f(pos_x, pos_y, pos_z, srcs, dsts)


_EBL = 1280


def _dense_body(vx_ref, vy_ref, vz_ref, StAT_ref, StBT_ref, W1sT_ref,
                W2sAT_ref, W2sBT_ref, oA_ref, oB_ref):
  x = vx_ref[0]
  y = vy_ref[0]
  zc = vz_ref[0]
  vl = jnp.sqrt(x * x + y * y + zc * zc + 1e-18)
  inv = 1.0 / vl
  ux = x * inv
  uy = y * inv
  uz = zc * inv
  s3 = math.sqrt(3.0)
  s5 = math.sqrt(5.0)
  x2 = ux * ux
  y2 = uy * uy
  z2 = uz * uz
  sh9T = jnp.concatenate([
      jnp.ones_like(vl),
      s3 * ux, s3 * uy, s3 * uz,
      s5 * (s3 * ux * uz),
      s5 * (s3 * ux * uy),
      s5 * (y2 - 0.5 * (x2 + z2)),
      s5 * (s3 * uy * uz),
      s5 * (0.5 * s3 * (z2 - x2)),
  ], axis=0)
  vlT = vl
  roots = ((lax.broadcasted_iota(jnp.int32, (NUM_BASIS, 1), 0).astype(_F32)
            + 1.0) * math.pi)
  safe_r = jnp.where(vlT > 1e-9, vlT, 1.0)
  mask = ((vlT < MAX_RADIUS) & (vlT > 0)).astype(_F32)
  coef = jnp.sqrt(2.0 / MAX_RADIUS) * mask / safe_r
  rbT = jnp.sin(roots * (vlT / MAX_RADIUS)) * coef
  hT = _SILU_CST * jax.nn.silu(
      jnp.dot(W1sT_ref[...], rbT, preferred_element_type=_F32))
  outAT = (jnp.dot(W2sAT_ref[...], hT, preferred_element_type=_F32) *
           jnp.dot(StAT_ref[...], sh9T, preferred_element_type=_F32))
  outBT = (jnp.dot(W2sBT_ref[...], hT, preferred_element_type=_F32) *
           jnp.dot(StBT_ref[...], sh9T, preferred_element_type=_F32))
  oA_ref[...] = outAT.T
  oB_ref[...] = outBT.T


def _edge_dense(vx, vy, vz, StAT, StBT, W1sT, W2sAT, W2sBT):
  grid = E // _EBL
  vx = vx.reshape(grid, 1, _EBL)
  vy = vy.reshape(grid, 1, _EBL)
  vz = vz.reshape(grid, 1, _EBL)
  return pl.pallas_call(
      _dense_body,
      grid=(grid,),
      in_specs=[
          pl.BlockSpec((1, 1, _EBL), lambda i: (i, 0, 0)),
          pl.BlockSpec((1, 1, _EBL), lambda i: (i, 0, 0)),
          pl.BlockSpec((1, 1, _EBL), lambda i: (i, 0, 0)),
          pl.BlockSpec((HALF, 9), lambda i: (0, 0)),
          pl.BlockSpec((HALF, 9), lambda i: (0, 0)),
          pl.BlockSpec((NUM_BASIS, NUM_BASIS), lambda i: (0, 0)),
          pl.BlockSpec((HALF, NUM_BASIS), lambda i: (0, 0)),
          pl.BlockSpec((HALF, NUM_BASIS), lambda i: (0, 0)),
      ],
      out_specs=[
          pl.BlockSpec((_EBL, HALF), lambda i: (i, 0)),
          pl.BlockSpec((_EBL, HALF), lambda i: (i, 0)),
      ],
      out_shape=[
          jax.ShapeDtypeStruct((E, HALF), _F32),
          jax.ShapeDtypeStruct((E, HALF), _F32),
      ],
  )(vx, vy, vz, StAT, StBT, W1sT, W2sAT, W2sBT)


_K = 40
_NCH = (E // 16) // _K
_EPS = E // 16


def _scatter_body(atomA, atomB, radA, radB, srcs, dsts, outA, outB,
                  acc, zb,
                  ss0, ds0, gb0, rb0, ss1, ds1, gb1, rb1,
                  sg0, sr0, sg1, sr1):
  cid = lax.axis_index("c")
  sid = lax.axis_index("s")

  @pl.loop(0, _K)
  def _(i):
    for k in range(HALF // 16):
      zb[i, pl.ds(k * 16, 16)] = jnp.zeros((16,), _F32)

  @pl.loop(0, NPAD // 16 // _K)
  def _(t):
    pltpu.sync_copy(zb, acc.at[pl.ds(sid * (NPAD // 16) + t * _K, _K)])
  plsc.subcore_barrier()

  base = sid * _EPS
  bufs = ((ss0, ds0, gb0, rb0, sg0, sr0), (ss1, ds1, gb1, rb1, sg1, sr1))

  def run(at_, rd_, ot_):
    def issue(g, bk):
      ss, dd, gb, rbf, sg, sr = bk
      off = base + g * _K
      pltpu.sync_copy(srcs.at[pl.ds(off, _K)], ss)
      pltpu.sync_copy(dsts.at[pl.ds(off, _K)], dd)
      pltpu.make_async_copy(at_.at[ss], gb, sg).start()
      pltpu.make_async_copy(rd_.at[pl.ds(off, _K)], rbf, sr).start()

    def process(g, bk):
      ss, dd, gb, rbf, sg, sr = bk
      off = base + g * _K
      pltpu.make_async_copy(at_.at[ss], gb, sg).wait()
      pltpu.make_async_copy(rd_.at[pl.ds(off, _K)], rbf, sr).wait()

      @pl.loop(0, _K)
      def _(i):
        for k in range(HALF // 16):
          sl = pl.ds(k * 16, 16)
          gb[i, sl] = gb[i, sl] * rbf[i, sl]

      pltpu.sync_copy(gb, acc.at[dd], add=True)

    issue(0, bufs[0])

    @pl.loop(0, _NCH - 2, step=2)
    def _(t):
      for b in range(2):
        g = t + b
        issue(g + 1, bufs[1 - b])
        process(g, bufs[b])

    issue(_NCH - 1, bufs[(_NCH - 1) % 2])
    process(_NCH - 2, bufs[(_NCH - 2) % 2])
    process(_NCH - 1, bufs[(_NCH - 1) % 2])
    plsc.subcore_barrier()

    @pl.loop(0, NPAD // 16 // _K)
    def _(t):
      row = sid * (NPAD // 16) + t * _K
      pltpu.sync_copy(acc.at[pl.ds(row, _K)], zb)
      pltpu.sync_copy(zb, ot_.at[pl.ds(row, _K)])

  @pl.when(cid == 0)
  def _():
    run(atomA, radA, outA)

  @pl.when(cid == 1)
  def _():
    run(atomB, radB, outB)


def _scatter(atomA, atomB, radA, radB, srcs, dsts):
  mesh = plsc.VectorSubcoreMesh(core_axis_name="c", subcore_axis_name="s")
  f = pl.kernel(
      _scatter_body,
      out_type=(
          jax.ShapeDtypeStruct((NPAD, HALF), _F32),
          jax.ShapeDtypeStruct((NPAD, HALF), _F32),
      ),
      mesh=mesh,
      scratch_types=[
          pltpu.VMEM_SHARED((NPAD, HALF), _F32),
          pltpu.VMEM((_K, HALF), _F32),
          pltpu.VMEM((_K,), jnp.int32),
          pltpu.VMEM((_K,), jnp.int32),
          pltpu.VMEM((_K, HALF), _F32),
          pltpu.VMEM((_K, HALF), _F32),
          pltpu.VMEM((_K,), jnp.int32),
          pltpu.VMEM((_K,), jnp.int32),
          pltpu.VMEM((_K, HALF), _F32),
          pltpu.VMEM((_K, HALF), _F32),
          pltpu.SemaphoreType.DMA,
          pltpu.SemaphoreType.DMA,
          pltpu.SemaphoreType.DMA,
          pltpu.SemaphoreType.DMA,
      ],
      compiler_params=pltpu.CompilerParams(use_tc_tiling_on_sc=False),
  )
  return f(atomA, atomB, radA, radB, srcs, dsts)


def kernel(pos, z, edge_index, W0, W1, W2, W_atom, b_atom, Wfc1, Wfc2):
  srcs = edge_index[0]
  dsts = edge_index[1]
  pos_x, pos_y, pos_z = pos[:, 0], pos[:, 1], pos[:, 2]

  St = jnp.zeros((9, DIM), _F32)
  St = St.at[0, 0:32].set(W0)
  St = St.at[1:4, 32:128].set(jnp.kron(W1[None, :], jnp.eye(3, dtype=_F32)))
  St = St.at[4:9, 128:288].set(jnp.kron(W2[None, :], jnp.eye(5, dtype=_F32)))
  StAT, StBT = St[:, :HALF].T, St[:, HALF:].T
  W1sT = (Wfc1 / math.sqrt(float(NUM_BASIS))).T
  W2s = Wfc2 / math.sqrt(32.0)
  W2sAT, W2sBT = W2s[:, :HALF].T, W2s[:, HALF:].T
  wt = W_atom.T
  wtA, wtB = wt[:, :HALF], wt[:, HALF:]
  bA, bB = b_atom[None, :HALF], b_atom[None, HALF:]

  atomA, atomB = _atom_tables(z, wtA, wtB, bA, bB)
  vx, vy, vz = _edge_vec(pos_x, pos_y, pos_z, srcs, dsts)
  radA, radB = _edge_dense(vx, vy, vz, StAT, StBT, W1sT, W2sAT, W2sBT)
  outA, outB = _scatter(atomA, atomB, radA, radB, srcs, dsts)
  return jnp.concatenate([outA[:N], outB[:N]], axis=1)
